# baseline (speedup 1.0000x reference)
import functools

import jax
import jax.numpy as jnp
from jax import lax
from jax.experimental import pallas as pl
from jax.experimental.pallas import tpu as pltpu
from jax.experimental.pallas import tpu_sc as plsc

NC = 2
NS = 16
NW = NC * NS
LANES = 16
K = 8
NRING = 6
UNROLL = 8


def _make_kernel(B, S, V, D):
    SW = S // NW
    CK = SW // K
    VPR = D // LANES
    JBLK = VPR // UNROLL
    HB = B // 2

    mesh = plsc.VectorSubcoreMesh(core_axis_name="c", subcore_axis_name="s")

    scratch = (
        [pltpu.VMEM((B * SW,), jnp.int32)]
        + [pltpu.VMEM((K, D), jnp.float32) for _ in range(NRING)]
        + [pltpu.VMEM((HB, D), jnp.float32) for _ in range(2)]
        + [pltpu.VMEM((K, D), jnp.float32)]
        + [pltpu.SemaphoreType.DMA for _ in range(NRING + 3)]
    )

    @functools.partial(
        pl.kernel,
        mesh=mesh,
        out_type=jax.ShapeDtypeStruct((B * S, D), jnp.float32),
        scratch_types=scratch,
    )
    def k(ids_hbm, word_hbm, pos_hbm, out_hbm, idx_all, *rest):
        o = rest[:NRING]
        st = rest[NRING:NRING + 2]
        pbuf = rest[NRING + 2]
        gsem = rest[NRING + 3:2 * NRING + 3]
        stsem = rest[2 * NRING + 3:2 * NRING + 5]
        psem = rest[2 * NRING + 5]

        wid = lax.axis_index("s") * NC + lax.axis_index("c")
        seq_base = wid * SW

        for b in range(B):
            pltpu.sync_copy(
                ids_hbm.at[pl.ds(b * S + seq_base, SW)],
                idx_all.at[pl.ds(b * SW, SW)],
            )

        def issue_gather(c, b):
            s = (B * c + b) % NRING
            pltpu.async_copy(
                word_hbm.at[idx_all.at[pl.ds(b * SW + c * K, K)]], o[s], gsem[s]
            )

        def wait_gather(c, b):
            s = (B * c + b) % NRING
            pltpu.make_async_copy(
                word_hbm.at[idx_all.at[pl.ds(b * SW + c * K, K)]], o[s], gsem[s]
            ).wait()

        def drain_staged():
            for h in range(2):
                pltpu.make_async_copy(
                    st[h], out_hbm.at[pl.ds(seq_base, HB)], stsem[h]
                ).wait()

        def issue_pos(c):
            pltpu.async_copy(
                pos_hbm.at[pl.ds(seq_base + c * K, K)], pbuf, psem
            )

        def wait_pos():
            pltpu.make_async_copy(
                pos_hbm.at[pl.ds(seq_base, K)], pbuf, psem
            ).wait()

        def fused_add(c, first):
            slots = [o[(B * c + b) % NRING] for b in range(B)]

            def row_body(r, _):
                if first:
                    @pl.when(r > 0)
                    def _():
                        drain_staged()
                else:
                    drain_staged()

                def col_body(j, _):
                    base = j * (LANES * UNROLL)
                    for u in range(UNROLL):
                        off = base + u * LANES
                        sl = pl.ds(off, LANES)
                        x = pbuf[r, sl]
                        for h in range(2):
                            for i in range(HB):
                                st[h][i, sl] = slots[h * HB + i][r, sl] + x
                    return 0
                lax.fori_loop(0, JBLK, col_body, 0)
                for h in range(2):
                    for i in range(HB):
                        b = h * HB + i
                        pltpu.async_copy(
                            st[h].at[pl.ds(i, 1)],
                            out_hbm.at[pl.ds(b * S + seq_base + c * K + r, 1)],
                            stsem[h],
                        )
                return 0
            lax.fori_loop(0, K, row_body, 0)

        issue_pos(0)
        for b in range(B):
            issue_gather(0, b)

        for c in range(CK):
            wait_pos()
            for b in range(B):
                wait_gather(c, b)
            if c + 1 < CK:
                issue_gather(c + 1, 0)
                issue_gather(c + 1, 1)
            fused_add(c, first=(c == 0))
            if c + 1 < CK:
                issue_pos(c + 1)
                issue_gather(c + 1, 2)
                issue_gather(c + 1, 3)

        drain_staged()

    return k


def kernel(input_ids, word_table, pos_table):
    B, S = input_ids.shape
    V, D = word_table.shape
    ids_flat = input_ids.reshape(B * S).astype(jnp.int32)
    k = _make_kernel(B, S, V, D)
    out = k(ids_flat, word_table, pos_table)
    return out.reshape(B, S, D)


# speedup vs baseline: 3.0616x; 3.0616x over previous
import functools

import jax
import jax.numpy as jnp
from jax import lax
from jax.experimental import pallas as pl
from jax.experimental.pallas import tpu as pltpu
from jax.experimental.pallas import tpu_sc as plsc

NC = 2
NS = 16
NW = NC * NS
LANES = 16
K = 8
NRING = 6
UNROLL = 8


def _make_kernel(B, S, V, D):
    SW = S // NW
    CK = SW // K
    VPR = D // LANES
    JBLK = VPR // UNROLL

    mesh = plsc.VectorSubcoreMesh(core_axis_name="c", subcore_axis_name="s")

    scratch = (
        [pltpu.VMEM((B * SW,), jnp.int32)]
        + [pltpu.VMEM((K, D), jnp.float32) for _ in range(NRING)]
        + [pltpu.VMEM((K, D), jnp.float32)]
        + [pltpu.SemaphoreType.DMA for _ in range(2 * NRING + 1)]
    )

    @functools.partial(
        pl.kernel,
        mesh=mesh,
        out_type=jax.ShapeDtypeStruct((B * S, D), jnp.float32),
        scratch_types=scratch,
    )
    def k(ids_hbm, word_hbm, pos_hbm, out_hbm, idx_all, *rest):
        o = rest[:NRING]
        pbuf = rest[NRING]
        gsem = rest[NRING + 1:2 * NRING + 1]
        wsem = rest[2 * NRING + 1:3 * NRING + 1]
        psem = rest[3 * NRING + 1]

        wid = lax.axis_index("s") * NC + lax.axis_index("c")
        seq_base = wid * SW

        for b in range(B):
            pltpu.sync_copy(
                ids_hbm.at[pl.ds(b * S + seq_base, SW)],
                idx_all.at[pl.ds(b * SW, SW)],
            )

        slot_busy = [None] * NRING

        def issue_gather(c, b):
            s = (B * c + b) % NRING
            prev = slot_busy[s]
            if prev is not None:
                pc, pb = prev
                pltpu.make_async_copy(
                    o[s],
                    out_hbm.at[pl.ds(pb * S + seq_base + pc * K, K)],
                    wsem[s],
                ).wait()
            pltpu.async_copy(
                word_hbm.at[idx_all.at[pl.ds(b * SW + c * K, K)]], o[s], gsem[s]
            )

        def wait_gather(c, b):
            s = (B * c + b) % NRING
            pltpu.make_async_copy(
                word_hbm.at[idx_all.at[pl.ds(b * SW + c * K, K)]], o[s], gsem[s]
            ).wait()

        def issue_row_writes(c, r):
            for b in range(B):
                s = (B * c + b) % NRING
                pltpu.async_copy(
                    o[s].at[pl.ds(r, 1)],
                    out_hbm.at[pl.ds(b * S + seq_base + c * K + r, 1)],
                    wsem[s],
                )

        def issue_pos(c):
            pltpu.async_copy(
                pos_hbm.at[pl.ds(seq_base + c * K, K)], pbuf, psem
            )

        def wait_pos():
            pltpu.make_async_copy(
                pos_hbm.at[pl.ds(seq_base, K)], pbuf, psem
            ).wait()

        def fused_add(c):
            slots = [o[(B * c + b) % NRING] for b in range(B)]

            def row_body(r, _):
                def col_body(j, _):
                    base = j * (LANES * UNROLL)
                    xs = [
                        pbuf[r, pl.ds(base + u * LANES, LANES)]
                        for u in range(UNROLL)
                    ]
                    for u in range(UNROLL):
                        off = base + u * LANES
                        for ov in slots:
                            plsc.addupdate(ov.at[r, pl.ds(off, LANES)], xs[u])
                    return 0
                lax.fori_loop(0, JBLK, col_body, 0)
                issue_row_writes(c, r)
                return 0
            lax.fori_loop(0, K, row_body, 0)
            for b in range(B):
                slot_busy[(B * c + b) % NRING] = (c, b)

        issue_pos(0)
        for b in range(B):
            issue_gather(0, b)

        for c in range(CK):
            wait_pos()
            for b in range(B):
                wait_gather(c, b)
            if c + 1 < CK:
                issue_gather(c + 1, 0)
                issue_gather(c + 1, 1)
            fused_add(c)
            if c + 1 < CK:
                issue_pos(c + 1)
                issue_gather(c + 1, 2)
                issue_gather(c + 1, 3)

        for s in range(NRING):
            if slot_busy[s] is not None:
                pltpu.make_async_copy(
                    o[s], out_hbm.at[pl.ds(seq_base, K)], wsem[s]
                ).wait()

    return k


def kernel(input_ids, word_table, pos_table):
    B, S = input_ids.shape
    V, D = word_table.shape
    ids_flat = input_ids.reshape(B * S).astype(jnp.int32)
    k = _make_kernel(B, S, V, D)
    out = k(ids_flat, word_table, pos_table)
    return out.reshape(B, S, D)


# speedup vs baseline: 3.3004x; 1.0780x over previous
import functools

import jax
import jax.numpy as jnp
from jax import lax
from jax.experimental import pallas as pl
from jax.experimental.pallas import tpu as pltpu
from jax.experimental.pallas import tpu_sc as plsc

NC = 2
NS = 16
NW = NC * NS
LANES = 16
K = 4
UNROLL = 8


def _make_kernel(B, S, V, D):
    SW = S // NW
    CK = SW // K
    VPR = D // LANES
    JBLK = VPR // UNROLL
    RPB = S // K

    mesh = plsc.VectorSubcoreMesh(core_axis_name="c", subcore_axis_name="s")

    scratch = (
        [pltpu.VMEM((B * CK, K), jnp.int32)]
        + [pltpu.VMEM((K, D), jnp.float32) for _ in range(2 * B)]
        + [pltpu.VMEM((K, D), jnp.float32) for _ in range(2)]
        + [pltpu.SemaphoreType.DMA for _ in range(4 * B + 2)]
    )

    @functools.partial(
        pl.kernel,
        mesh=mesh,
        out_type=jax.ShapeDtypeStruct((B * S, D), jnp.float32),
        scratch_types=scratch,
    )
    def k(ids_hbm, word_hbm, pos_hbm, out_hbm, idx_all, *rest):
        o = rest[:2 * B]
        pbuf = rest[2 * B:2 * B + 2]
        gsem = rest[2 * B + 2:4 * B + 2]
        wsem = rest[4 * B + 2:6 * B + 2]
        psem = rest[6 * B + 2:6 * B + 4]

        wid = lax.axis_index("s") * NC + lax.axis_index("c")
        seq_base = wid * SW

        for b in range(B):
            pltpu.sync_copy(
                ids_hbm.at[pl.ds(b * RPB + wid * CK, CK)],
                idx_all.at[pl.ds(b * CK, CK)],
            )

        def issue_gather(c, b, q):
            s = B * q + b
            pltpu.async_copy(
                word_hbm.at[idx_all.at[b * CK + c]], o[s], gsem[s]
            )

        def wait_gather(c, b, q):
            s = B * q + b
            pltpu.make_async_copy(
                word_hbm.at[idx_all.at[b * CK + c]], o[s], gsem[s]
            ).wait()

        def drain_writes(b, q):
            s = B * q + b
            pltpu.make_async_copy(
                o[s], out_hbm.at[pl.ds(seq_base, K)], wsem[s]
            ).wait()

        def issue_pos(c, q):
            pltpu.async_copy(
                pos_hbm.at[pl.ds(seq_base + c * K, K)], pbuf[q], psem[q]
            )

        def wait_pos(q):
            pltpu.make_async_copy(
                pos_hbm.at[pl.ds(seq_base, K)], pbuf[q], psem[q]
            ).wait()

        def fused_add(c, q):
            slots = [o[B * q + b] for b in range(B)]
            pb = pbuf[q]

            def row_body(r, _):
                def col_body(j, _):
                    base = j * (LANES * UNROLL)
                    xs = [
                        pb[r, pl.ds(base + u * LANES, LANES)]
                        for u in range(UNROLL)
                    ]
                    for u in range(UNROLL):
                        off = base + u * LANES
                        for ov in slots:
                            plsc.addupdate(ov.at[r, pl.ds(off, LANES)], xs[u])
                    return 0
                lax.fori_loop(0, JBLK, col_body, 0)
                for b in range(B):
                    s = B * q + b
                    pltpu.async_copy(
                        o[s].at[pl.ds(r, 1)],
                        out_hbm.at[pl.ds(b * S + seq_base + c * K + r, 1)],
                        wsem[s],
                    )
                return 0
            lax.fori_loop(0, K, row_body, 0)

        issue_pos(0, 0)
        issue_pos(1, 1)
        for b in range(B):
            issue_gather(0, b, 0)
        for b in range(B):
            issue_gather(1, b, 1)

        wait_pos(0)
        for b in range(B):
            wait_gather(0, b, 0)
        fused_add(0, 0)
        issue_pos(2, 0)

        wait_pos(1)
        for b in range(B):
            wait_gather(1, b, 1)
        for b in range(B):
            drain_writes(b, 0)
            issue_gather(2, b, 0)
        fused_add(1, 1)
        issue_pos(3, 1)

        def do_chunk(c, q):
            wait_pos(q)
            for b in range(B):
                wait_gather(c, b, q)
            @pl.when(c + 1 < CK)
            def _():
                for b in range(B):
                    drain_writes(b, 1 - q)
                    issue_gather(c + 1, b, 1 - q)
            fused_add(c, q)

            @pl.when(c + 2 < CK)
            def _():
                issue_pos(c + 2, q)

        def step_body(s2, _):
            do_chunk(2 * s2, 0)
            do_chunk(2 * s2 + 1, 1)
            return 0

        lax.fori_loop(1, CK // 2, step_body, 0)

        for s in range(2 * B):
            pltpu.make_async_copy(
                o[s], out_hbm.at[pl.ds(seq_base, K)], wsem[s]
            ).wait()

    return k


def kernel(input_ids, word_table, pos_table):
    B, S = input_ids.shape
    V, D = word_table.shape
    ids2 = input_ids.reshape((B * S) // K, K).astype(jnp.int32)
    k = _make_kernel(B, S, V, D)
    out = k(ids2, word_table, pos_table)
    return out.reshape(B, S, D)


# speedup vs baseline: 3.3499x; 1.0150x over previous
import functools

import jax
import jax.numpy as jnp
from jax import lax
from jax.experimental import pallas as pl
from jax.experimental.pallas import tpu as pltpu
from jax.experimental.pallas import tpu_sc as plsc

NC = 2
NS = 16
NW = NC * NS
LANES = 16
K = 4
UNROLL = 8


def _make_kernel(B, S, V, D):
    SW = S // NW
    CK = SW // K
    VPR = D // LANES
    JBLK = VPR // UNROLL
    RPB = S // K

    mesh = plsc.VectorSubcoreMesh(core_axis_name="c", subcore_axis_name="s")

    scratch = (
        [pltpu.VMEM((B * CK, K), jnp.int32)]
        + [pltpu.VMEM((K, D), jnp.float32) for _ in range(2 * B)]
        + [pltpu.VMEM((K, D), jnp.float32) for _ in range(2)]
        + [pltpu.SemaphoreType.DMA for _ in range(4 * B + 2)]
    )

    @functools.partial(
        pl.kernel,
        mesh=mesh,
        out_type=jax.ShapeDtypeStruct((B * S, D), jnp.float32),
        scratch_types=scratch,
    )
    def k(ids_hbm, word_hbm, pos_hbm, out_hbm, idx_all, *rest):
        o = rest[:2 * B]
        pbuf = rest[2 * B:2 * B + 2]
        gsem = rest[2 * B + 2:4 * B + 2]
        wsem = rest[4 * B + 2:6 * B + 2]
        psem = rest[6 * B + 2:6 * B + 4]

        wid = lax.axis_index("s") * NC + lax.axis_index("c")
        seq_base = wid * SW

        for b in range(B):
            pltpu.async_copy(
                ids_hbm.at[pl.ds(b * RPB + wid * CK, CK)],
                idx_all.at[pl.ds(b * CK, CK)],
                psem[0],
            )
        for b in range(B):
            pltpu.make_async_copy(
                ids_hbm.at[pl.ds(b * RPB + wid * CK, CK)],
                idx_all.at[pl.ds(b * CK, CK)],
                psem[0],
            ).wait()

        def issue_gather(c, b, q):
            s = B * q + b
            pltpu.async_copy(
                word_hbm.at[idx_all.at[b * CK + c]], o[s], gsem[s]
            )

        def wait_gather(c, b, q):
            s = B * q + b
            pltpu.make_async_copy(
                word_hbm.at[idx_all.at[b * CK + c]], o[s], gsem[s]
            ).wait()

        def drain_writes(b, q):
            s = B * q + b
            pltpu.make_async_copy(
                o[s], out_hbm.at[pl.ds(seq_base, K)], wsem[s]
            ).wait()

        def issue_pos(c, q):
            pltpu.async_copy(
                pos_hbm.at[pl.ds(seq_base + c * K, K)], pbuf[q], psem[q]
            )

        def wait_pos(q):
            pltpu.make_async_copy(
                pos_hbm.at[pl.ds(seq_base, K)], pbuf[q], psem[q]
            ).wait()

        def fused_add(c, q):
            slots = [o[B * q + b] for b in range(B)]
            pb = pbuf[q]

            def row_body(r, _):
                def col_body(j, _):
                    base = j * (LANES * UNROLL)
                    xs = [
                        pb[r, pl.ds(base + u * LANES, LANES)]
                        for u in range(UNROLL)
                    ]
                    for u in range(UNROLL):
                        off = base + u * LANES
                        for ov in slots:
                            plsc.addupdate(ov.at[r, pl.ds(off, LANES)], xs[u])
                    return 0
                lax.fori_loop(0, JBLK, col_body, 0)
                for b in range(B):
                    s = B * q + b
                    pltpu.async_copy(
                        o[s].at[pl.ds(r, 1)],
                        out_hbm.at[pl.ds(b * S + seq_base + c * K + r, 1)],
                        wsem[s],
                    )
                return 0
            lax.fori_loop(0, K, row_body, 0)

        issue_pos(0, 0)
        issue_pos(1, 1)
        for b in range(B):
            issue_gather(0, b, 0)
        for b in range(B):
            issue_gather(1, b, 1)

        wait_pos(0)
        for b in range(B):
            wait_gather(0, b, 0)
        fused_add(0, 0)
        issue_pos(2, 0)

        wait_pos(1)
        for b in range(B):
            wait_gather(1, b, 1)
        for b in range(B):
            drain_writes(b, 0)
            issue_gather(2, b, 0)
        fused_add(1, 1)
        issue_pos(3, 1)

        def do_chunk(c, q):
            wait_pos(q)
            for b in range(B):
                wait_gather(c, b, q)
            @pl.when(c + 1 < CK)
            def _():
                for b in range(B):
                    drain_writes(b, 1 - q)
                    issue_gather(c + 1, b, 1 - q)
            fused_add(c, q)

            @pl.when(c + 2 < CK)
            def _():
                issue_pos(c + 2, q)

        def step_body(s2, _):
            do_chunk(2 * s2, 0)
            do_chunk(2 * s2 + 1, 1)
            return 0

        lax.fori_loop(1, CK // 2, step_body, 0)

        for s in range(2 * B):
            pltpu.make_async_copy(
                o[s], out_hbm.at[pl.ds(seq_base, K)], wsem[s]
            ).wait()

    return k


def kernel(input_ids, word_table, pos_table):
    B, S = input_ids.shape
    V, D = word_table.shape
    ids2 = input_ids.reshape((B * S) // K, K).astype(jnp.int32)
    k = _make_kernel(B, S, V, D)
    out = k(ids2, word_table, pos_table)
    return out.reshape(B, S, D)
